# baseline (speedup 1.0000x reference)
import functools

import jax
import jax.numpy as jnp
from jax import lax
from jax.experimental import pallas as pl
from jax.experimental.pallas import tpu as pltpu
from jax.experimental.pallas import tpu_sc as plsc

_EMBED = 64
_BATCH = 16384
_HIST = 50
_B = _BATCH * _HIST

_NC = 2
_NS = 16
_NW = _NC * _NS
_C = 128
_ROWS_PER_W = _B // _NW
_N_CHUNKS = _ROWS_PER_W // _C


def _build_kernel():
    mesh = plsc.VectorSubcoreMesh(core_axis_name="c", subcore_axis_name="s")

    NBUF = 8

    @functools.partial(
        pl.kernel,
        mesh=mesh,
        out_type=jax.ShapeDtypeStruct((_B, _EMBED), jnp.float32),
        scratch_types=[
            pltpu.VMEM((_N_CHUNKS, _C), jnp.int32),
            pltpu.VMEM((NBUF, _C, _EMBED), jnp.float32),
            pltpu.SemaphoreType.DMA((NBUF,)),
            pltpu.SemaphoreType.DMA((NBUF,)),
        ],
        compiler_params=pltpu.CompilerParams(use_tc_tiling_on_sc=False),
    )
    def gather_kernel(tok_hbm, tab_hbm, out_hbm, idx_v, rows_v, gsem, osem):
        wid = lax.axis_index("s") * _NC + lax.axis_index("c")
        base = wid * _ROWS_PER_W
        pltpu.sync_copy(tok_hbm.at[wid], idx_v)

        for b in range(NBUF - 1):
            pltpu.async_copy(tab_hbm.at[idx_v.at[b]], rows_v.at[b], gsem.at[b])

        def step(j, carry):
            b = lax.rem(j, NBUF)
            jn = j + NBUF - 1
            bn = lax.rem(jn, NBUF)

            @pl.when(jn < _N_CHUNKS)
            def _start_gather():
                @pl.when(jn >= NBUF)
                def _wait_out():
                    pltpu.make_async_copy(
                        rows_v.at[bn], out_hbm.at[pl.ds(0, _C)], osem.at[bn]
                    ).wait()

                pltpu.async_copy(tab_hbm.at[idx_v.at[jn]], rows_v.at[bn], gsem.at[bn])

            pltpu.make_async_copy(
                tab_hbm.at[pl.ds(0, _C)], rows_v.at[b], gsem.at[b]
            ).wait()
            pltpu.async_copy(rows_v.at[b], out_hbm.at[pl.ds(base + j * _C, _C)], osem.at[b])
            return carry

        lax.fori_loop(0, _N_CHUNKS, step, 0)

        for i in range(NBUF):
            b = (_N_CHUNKS - NBUF + i) % NBUF
            pltpu.make_async_copy(
                rows_v.at[b], out_hbm.at[pl.ds(0, _C)], osem.at[b]
            ).wait()

    return gather_kernel


_KERNEL = _build_kernel()

_VOCAB1 = 1000001
_TCHUNK = 512


def _tc_transpose(embT):
    grid = (pl.cdiv(_VOCAB1, _TCHUNK),)
    rows_per_blk = _TCHUNK * _EMBED // 128

    def body(in_ref, out_ref):
        t = jnp.swapaxes(in_ref[...], 0, 1)
        p = t.reshape(rows_per_blk, 2, _EMBED)
        out_ref[...] = jnp.concatenate([p[:, 0, :], p[:, 1, :]], axis=1)

    n_blk = pl.cdiv(_VOCAB1, _TCHUNK)
    return pl.pallas_call(
        body,
        grid=grid,
        in_specs=[pl.BlockSpec((_EMBED, _TCHUNK), lambda j: (0, j))],
        out_specs=pl.BlockSpec((rows_per_blk, 128), lambda j: (j, 0)),
        out_shape=jax.ShapeDtypeStruct((n_blk * rows_per_blk, 128), jnp.float32),
    )(embT)


def kernel(token, embed):
    tok = token.reshape(_NW, _N_CHUNKS, _C).astype(jnp.int32)
    tab2d = _tc_transpose(embed.T)
    tab_rows = tab2d.shape[0] * 128 // _EMBED
    out = _KERNEL(tok, tab2d.reshape(tab_rows, _EMBED))
    return lax.stop_gradient(out.reshape(_BATCH, _HIST, _EMBED))


# speedup vs baseline: 2.5472x; 2.5472x over previous
import functools

import jax
import jax.numpy as jnp
from jax import lax
from jax.experimental import pallas as pl
from jax.experimental.pallas import tpu as pltpu
from jax.experimental.pallas import tpu_sc as plsc

_EMBED = 64
_BATCH = 16384
_HIST = 50
_B = _BATCH * _HIST

_NC = 2
_NS = 16
_NW = _NC * _NS
_BPC = 2
_C = _BPC * _HIST
_N_CHUNKS = _BATCH // (_NW * _BPC)
_HPAD = 56
_EPAD = 128

_NBUF = 8


def _build_kernel():
    mesh = plsc.VectorSubcoreMesh(core_axis_name="c", subcore_axis_name="s")

    @functools.partial(
        pl.kernel,
        mesh=mesh,
        out_type=jax.ShapeDtypeStruct((_BATCH, _HPAD, _EPAD), jnp.float32),
        scratch_types=[
            pltpu.VMEM((_N_CHUNKS, _C), jnp.int32),
            pltpu.VMEM((_NBUF, _C, _EMBED), jnp.float32),
            pltpu.SemaphoreType.DMA((_NBUF,)),
            pltpu.SemaphoreType.DMA((_NBUF,)),
        ],
        compiler_params=pltpu.CompilerParams(
            use_tc_tiling_on_sc=False, needs_layout_passes=False
        ),
    )
    def gather_kernel(tok_hbm, tab_hbm, out_hbm, idx_v, gbuf, gsem, osem):
        wid = lax.axis_index("s") * _NC + lax.axis_index("c")
        bbase = wid * _N_CHUNKS * _BPC
        pltpu.sync_copy(tok_hbm.at[wid], idx_v)

        for b in range(_NBUF - 1):
            pltpu.async_copy(tab_hbm.at[idx_v.at[b]], gbuf.at[b], gsem.at[b])

        def step(j, carry):
            b = lax.rem(j, _NBUF)
            jn = j + _NBUF - 1
            bn = lax.rem(jn, _NBUF)

            @pl.when(jn < _N_CHUNKS)
            def _start_gather():
                @pl.when(jn >= _NBUF)
                def _wait_out():
                    for q in range(_BPC):
                        pltpu.make_async_copy(
                            gbuf.at[bn, pl.ds(q * _HIST, _HIST)],
                            out_hbm.at[0, pl.ds(0, _HIST), pl.ds(0, _EMBED)],
                            osem.at[bn],
                        ).wait()

                pltpu.async_copy(tab_hbm.at[idx_v.at[jn]], gbuf.at[bn], gsem.at[bn])

            pltpu.make_async_copy(
                tab_hbm.at[pl.ds(0, _C)], gbuf.at[b], gsem.at[b]
            ).wait()
            bi = bbase + j * _BPC
            for q in range(_BPC):
                pltpu.async_copy(
                    gbuf.at[b, pl.ds(q * _HIST, _HIST)],
                    out_hbm.at[bi + q, pl.ds(0, _HIST), pl.ds(0, _EMBED)],
                    osem.at[b],
                )
            return carry

        lax.fori_loop(0, _N_CHUNKS, step, 0)

        for i in range(_NBUF):
            b = (_N_CHUNKS - _NBUF + i) % _NBUF
            for q in range(_BPC):
                pltpu.make_async_copy(
                    gbuf.at[b, pl.ds(q * _HIST, _HIST)],
                    out_hbm.at[0, pl.ds(0, _HIST), pl.ds(0, _EMBED)],
                    osem.at[b],
                ).wait()

    return gather_kernel


_KERNEL = _build_kernel()

_VOCAB1 = 1000001
_TCHUNK = 8192


def _tc_transpose(embT):
    grid = (pl.cdiv(_VOCAB1, _TCHUNK),)
    rows_per_blk = _TCHUNK * _EMBED // 128

    def body(in_ref, out_ref):
        t = jnp.swapaxes(in_ref[...], 0, 1)
        p = t.reshape(rows_per_blk, 2, _EMBED)
        out_ref[...] = jnp.concatenate([p[:, 0, :], p[:, 1, :]], axis=1)

    n_blk = pl.cdiv(_VOCAB1, _TCHUNK)
    return pl.pallas_call(
        body,
        grid=grid,
        in_specs=[pl.BlockSpec((_EMBED, _TCHUNK), lambda j: (0, j))],
        out_specs=pl.BlockSpec((rows_per_blk, 128), lambda j: (j, 0)),
        out_shape=jax.ShapeDtypeStruct((n_blk * rows_per_blk, 128), jnp.float32),
    )(embT)


def kernel(token, embed):
    tok = token.reshape(_NW, _N_CHUNKS, _C).astype(jnp.int32)
    tab2d = _tc_transpose(embed.T)
    tab_rows = tab2d.shape[0] * 128 // _EMBED
    out4 = _KERNEL(tok, tab2d.reshape(tab_rows, _EMBED))
    return lax.stop_gradient(out4[:, :_HIST, :_EMBED])


# speedup vs baseline: 2.5557x; 1.0033x over previous
import functools

import jax
import jax.numpy as jnp
from jax import lax
from jax.experimental import pallas as pl
from jax.experimental.pallas import tpu as pltpu
from jax.experimental.pallas import tpu_sc as plsc

_EMBED = 64
_BATCH = 16384
_HIST = 50
_B = _BATCH * _HIST

_NC = 2
_NS = 16
_NW = _NC * _NS
_BPC = 2
_C = _BPC * _HIST
_N_CHUNKS = _BATCH // (_NW * _BPC)
_HPAD = 56
_EPAD = 128

_NBUF = 8


def _build_kernel():
    mesh = plsc.VectorSubcoreMesh(core_axis_name="c", subcore_axis_name="s")

    @functools.partial(
        pl.kernel,
        mesh=mesh,
        out_type=jax.ShapeDtypeStruct((_BATCH, _HPAD, _EPAD), jnp.float32),
        scratch_types=[
            pltpu.VMEM((_N_CHUNKS, _C), jnp.int32),
            pltpu.VMEM((_NBUF, _C, _EMBED), jnp.float32),
            pltpu.SemaphoreType.DMA((_NBUF,)),
            pltpu.SemaphoreType.DMA((_NBUF,)),
        ],
        compiler_params=pltpu.CompilerParams(
            use_tc_tiling_on_sc=False, needs_layout_passes=False
        ),
    )
    def gather_kernel(tok_hbm, tab_hbm, out_hbm, idx_v, gbuf, gsem, osem):
        wid = lax.axis_index("s") * _NC + lax.axis_index("c")
        bbase = wid * _N_CHUNKS * _BPC
        pltpu.sync_copy(tok_hbm.at[wid], idx_v)

        for b in range(_NBUF - 1):
            pltpu.async_copy(tab_hbm.at[idx_v.at[b]], gbuf.at[b], gsem.at[b])

        def step(j, carry):
            b = lax.rem(j, _NBUF)
            jn = j + _NBUF - 1
            bn = lax.rem(jn, _NBUF)

            @pl.when(jn < _N_CHUNKS)
            def _start_gather():
                @pl.when(jn >= _NBUF)
                def _wait_out():
                    for q in range(_BPC):
                        pltpu.make_async_copy(
                            gbuf.at[bn, pl.ds(q * _HIST, _HIST)],
                            out_hbm.at[0, pl.ds(0, _HIST), pl.ds(0, _EMBED)],
                            osem.at[bn],
                        ).wait()

                pltpu.async_copy(tab_hbm.at[idx_v.at[jn]], gbuf.at[bn], gsem.at[bn])

            pltpu.make_async_copy(
                tab_hbm.at[pl.ds(0, _C)], gbuf.at[b], gsem.at[b]
            ).wait()
            bi = bbase + j * _BPC
            for q in range(_BPC):
                pltpu.async_copy(
                    gbuf.at[b, pl.ds(q * _HIST, _HIST)],
                    out_hbm.at[bi + q, pl.ds(0, _HIST), pl.ds(0, _EMBED)],
                    osem.at[b],
                )
            return carry

        lax.fori_loop(0, _N_CHUNKS, step, 0)

        for i in range(_NBUF):
            b = (_N_CHUNKS - _NBUF + i) % _NBUF
            for q in range(_BPC):
                pltpu.make_async_copy(
                    gbuf.at[b, pl.ds(q * _HIST, _HIST)],
                    out_hbm.at[0, pl.ds(0, _HIST), pl.ds(0, _EMBED)],
                    osem.at[b],
                ).wait()

    return gather_kernel


_KERNEL = _build_kernel()

_VOCAB1 = 1000001
_TCHUNK = 16384


def _tc_transpose(embT):
    grid = (pl.cdiv(_VOCAB1, _TCHUNK),)
    rows_per_blk = _TCHUNK * _EMBED // 128

    def body(in_ref, out_ref):
        t = jnp.swapaxes(in_ref[...], 0, 1)
        p = t.reshape(rows_per_blk, 2, _EMBED)
        out_ref[...] = jnp.concatenate([p[:, 0, :], p[:, 1, :]], axis=1)

    n_blk = pl.cdiv(_VOCAB1, _TCHUNK)
    return pl.pallas_call(
        body,
        grid=grid,
        in_specs=[pl.BlockSpec((_EMBED, _TCHUNK), lambda j: (0, j))],
        out_specs=pl.BlockSpec((rows_per_blk, 128), lambda j: (j, 0)),
        out_shape=jax.ShapeDtypeStruct((n_blk * rows_per_blk, 128), jnp.float32),
    )(embT)


def kernel(token, embed):
    tok = token.reshape(_NW, _N_CHUNKS, _C).astype(jnp.int32)
    tab2d = _tc_transpose(embed.T)
    tab_rows = tab2d.shape[0] * 128 // _EMBED
    out4 = _KERNEL(tok, tab2d.reshape(tab_rows, _EMBED))
    return lax.stop_gradient(out4[:, :_HIST, :_EMBED])
